# baseline (speedup 1.0000x reference)
import math

import jax
import jax.numpy as jnp
from jax import lax
from jax.experimental import pallas as pl
from jax.experimental.pallas import tpu as pltpu
from jax.experimental.pallas import tpu_sc as plsc

VOCAB = 100000
EMBED = 128
SEQ_LEN = 2048
BATCH = 4
ROWS = SEQ_LEN * BATCH
NC, NS, LANES = 2, 16, 16
NW = NC * NS
RPW = ROWS // NW
PE_RPW = RPW // BATCH
CHUNKS = EMBED // LANES
SCALE = math.sqrt(EMBED)

_mesh = plsc.VectorSubcoreMesh(
    core_axis_name="c", subcore_axis_name="s", num_cores=NC, num_subcores=NS
)


@pl.kernel(
    out_type=jax.ShapeDtypeStruct((ROWS, EMBED), jnp.float32),
    mesh=_mesh,
    scratch_types=[
        pltpu.VMEM((2, 128), jnp.int32),
        pltpu.VMEM((2, 128), jnp.int32),
        pltpu.VMEM((RPW, EMBED), jnp.float32),
        pltpu.VMEM((RPW, EMBED), jnp.float32),
        pltpu.VMEM((PE_RPW, EMBED), jnp.float32),
        pltpu.SemaphoreType.DMA,
    ],
)
def _sc_embed(seq_hbm, seg_hbm, tok_table, seg_table, pe_hbm, out_hbm,
              idx_v, lbl_v, tok_v, seg_v, pe_v, sem):
    wid = lax.axis_index("s") * NC + lax.axis_index("c")
    base = wid * RPW

    pltpu.sync_copy(seq_hbm.at[pl.ds(wid * 2, 2)], idx_v)
    pltpu.sync_copy(seg_hbm.at[pl.ds(wid * 2, 2)], lbl_v)

    cps = []
    for h in range(2):
        cps.append(pltpu.async_copy(
            tok_table.at[idx_v.at[h]],
            tok_v.at[pl.ds(h * 128, 128)], sem))
        cps.append(pltpu.async_copy(
            seg_table.at[lbl_v.at[h]],
            seg_v.at[pl.ds(h * 128, 128)], sem))
    cps.append(pltpu.async_copy(
        pe_hbm.at[pl.ds(wid * PE_RPW, PE_RPW)], pe_v, sem))
    for cp in cps:
        cp.wait()

    def body(p, _):
        for q in range(BATCH):
            r = p * BATCH + q
            for c in range(CHUNKS):
                sl = pl.ds(c * LANES, LANES)
                chunk = (tok_v[r, sl] * SCALE + pe_v[p, sl] + seg_v[r, sl])
                tok_v[r, sl] = chunk
        return _

    lax.fori_loop(0, PE_RPW, body, 0, unroll=False)

    pltpu.sync_copy(tok_v, out_hbm.at[pl.ds(base, RPW)])


def kernel(sequence, segment_label, token_table, segment_table, pe):
    seq2d = sequence.reshape(ROWS // 128, 128).astype(jnp.int32)
    lbl2d = segment_label.reshape(ROWS // 128, 128).astype(jnp.int32)
    pe2d = pe[:SEQ_LEN].reshape(SEQ_LEN, EMBED)
    out = _sc_embed(seq2d, lbl2d, token_table, segment_table, pe2d)
    return out.reshape(SEQ_LEN, BATCH, EMBED)


# speedup vs baseline: 3.2134x; 3.2134x over previous
import math

import jax
import jax.numpy as jnp
from jax import lax
from jax.experimental import pallas as pl
from jax.experimental.pallas import tpu as pltpu
from jax.experimental.pallas import tpu_sc as plsc

VOCAB = 100000
EMBED = 128
SEQ_LEN = 2048
BATCH = 4
ROWS = SEQ_LEN * BATCH
NC, NS, LANES = 2, 16, 16
NW = NC * NS
RPW = ROWS // NW
PE_RPW = RPW // BATCH
CHUNKS = EMBED // LANES
HALF = RPW // 2
SCALE = math.sqrt(EMBED)

_mesh = plsc.VectorSubcoreMesh(
    core_axis_name="c", subcore_axis_name="s", num_cores=NC, num_subcores=NS
)


@pl.kernel(
    out_type=jax.ShapeDtypeStruct((ROWS, EMBED), jnp.float32),
    mesh=_mesh,
    scratch_types=[
        pltpu.VMEM((2, 128), jnp.int32),
        pltpu.VMEM((2, 128), jnp.int32),
        pltpu.VMEM((RPW, EMBED), jnp.float32),
        pltpu.VMEM((4, EMBED), jnp.float32),
        pltpu.VMEM((PE_RPW, EMBED), jnp.float32),
        pltpu.SemaphoreType.DMA,
        pltpu.SemaphoreType.DMA,
        pltpu.SemaphoreType.DMA,
    ],
)
def _sc_embed(seq_hbm, seg_hbm, tok_table, seg_table, pe_hbm, out_hbm,
              idx_v, lbl_v, tok_v, segt_v, pe_v, sem0, sem1, semw):
    wid = lax.axis_index("s") * NC + lax.axis_index("c")
    base = wid * RPW

    pltpu.sync_copy(seq_hbm.at[pl.ds(wid * 2, 2)], idx_v)
    pltpu.sync_copy(seg_hbm.at[pl.ds(wid * 2, 2)], lbl_v)

    g0 = pltpu.async_copy(tok_table.at[idx_v.at[0]],
                          tok_v.at[pl.ds(0, HALF)], sem0)
    g1 = pltpu.async_copy(tok_table.at[idx_v.at[1]],
                          tok_v.at[pl.ds(HALF, HALF)], sem1)
    gs = pltpu.async_copy(seg_table.at[pl.ds(0, 3)], segt_v.at[pl.ds(0, 3)],
                          sem0)
    gp = pltpu.async_copy(pe_hbm.at[pl.ds(wid * PE_RPW, PE_RPW)], pe_v, sem0)
    g0.wait(); gs.wait(); gp.wait()

    seg1 = [segt_v[1, pl.ds(c * LANES, LANES)] for c in range(CHUNKS)]
    seg2 = [segt_v[2, pl.ds(c * LANES, LANES)] for c in range(CHUNKS)]

    wb = []

    for h in range(2):
        if h == 1:
            g1.wait()

        def block(kk, _, h=h):
            lblv = lbl_v[h, pl.ds(kk * LANES, LANES)]
            for i in range(LANES):
                r = (h * 8 + kk) * LANES + i
                lbl_b = lax.gather(
                    lblv,
                    jnp.full((LANES, 1), i, jnp.int32),
                    lax.GatherDimensionNumbers(
                        offset_dims=(), collapsed_slice_dims=(0,),
                        start_index_map=(0,)),
                    slice_sizes=(1,),
                    mode=lax.GatherScatterMode.PROMISE_IN_BOUNDS)
                f1 = (lbl_b & 1).astype(jnp.float32)
                f2 = (lbl_b >> 1).astype(jnp.float32)
                p = r // BATCH
                for c in range(CHUNKS):
                    sl = pl.ds(c * LANES, LANES)
                    tok_v[r, sl] = (tok_v[r, sl] * SCALE + pe_v[p, sl]
                                    + f1 * seg1[c] + f2 * seg2[c])
            return _

        lax.fori_loop(0, 8, block, 0, unroll=False)

        wb.append(pltpu.async_copy(
            tok_v.at[pl.ds(h * HALF, HALF)],
            out_hbm.at[pl.ds(base + h * HALF, HALF)], semw))

    for cp in wb:
        cp.wait()


def kernel(sequence, segment_label, token_table, segment_table, pe):
    seq2d = sequence.reshape(ROWS // 128, 128).astype(jnp.int32)
    lbl2d = segment_label.reshape(ROWS // 128, 128).astype(jnp.int32)
    pe2d = pe[:SEQ_LEN].reshape(SEQ_LEN, EMBED)
    out = _sc_embed(seq2d, lbl2d, token_table, segment_table, pe2d)
    return out.reshape(SEQ_LEN, BATCH, EMBED)


# speedup vs baseline: 3.7747x; 1.1747x over previous
import math

import jax
import jax.numpy as jnp
from jax import lax
from jax.experimental import pallas as pl
from jax.experimental.pallas import tpu as pltpu
from jax.experimental.pallas import tpu_sc as plsc

VOCAB = 100000
EMBED = 128
SEQ_LEN = 2048
BATCH = 4
ROWS = SEQ_LEN * BATCH
NC, NS, LANES = 2, 16, 16
NW = NC * NS
RPW = ROWS // NW
PE_RPW = RPW // BATCH
CHUNKS = EMBED // LANES
NQ = 4
QROWS = RPW // NQ
SCALE = math.sqrt(EMBED)

_mesh = plsc.VectorSubcoreMesh(
    core_axis_name="c", subcore_axis_name="s", num_cores=NC, num_subcores=NS
)


@pl.kernel(
    out_type=jax.ShapeDtypeStruct((ROWS, EMBED), jnp.float32),
    mesh=_mesh,
    scratch_types=[
        pltpu.VMEM((NQ, QROWS), jnp.int32),
        pltpu.VMEM((NQ, QROWS), jnp.int32),
        pltpu.VMEM((RPW, EMBED), jnp.float32),
        pltpu.VMEM((RPW, EMBED), jnp.float32),
        pltpu.VMEM((4, EMBED), jnp.float32),
        pltpu.VMEM((PE_RPW, EMBED), jnp.float32),
        pltpu.SemaphoreType.DMA,
        pltpu.SemaphoreType.DMA,
        pltpu.SemaphoreType.DMA,
        pltpu.SemaphoreType.DMA,
        pltpu.SemaphoreType.DMA,
        pltpu.SemaphoreType.DMA,
    ],
)
def _sc_embed(seq_hbm, seg_hbm, tok_table, seg_table, pe_hbm, out_hbm,
              idx_v, lbl_v, tok_v, out_v, segt_v, pe_v,
              sems, semg0, semg1, semg2, semg3, semw):
    wid = lax.axis_index("s") * NC + lax.axis_index("c")
    base = wid * RPW
    semg = [semg0, semg1, semg2, semg3]

    cpi = pltpu.async_copy(seq_hbm.at[pl.ds(wid * NQ, NQ)], idx_v, sems)
    cpl = pltpu.async_copy(seg_hbm.at[pl.ds(wid * NQ, NQ)], lbl_v, sems)
    cpi.wait()

    gq = [pltpu.async_copy(tok_table.at[idx_v.at[q]],
                           tok_v.at[pl.ds(q * QROWS, QROWS)], semg[q])
          for q in range(NQ)]
    gs = pltpu.async_copy(seg_table.at[pl.ds(0, 3)], segt_v.at[pl.ds(0, 3)],
                          sems)
    gp = pltpu.async_copy(pe_hbm.at[pl.ds(wid * PE_RPW, PE_RPW)], pe_v, sems)
    cpl.wait(); gs.wait(); gp.wait()

    seg1 = [segt_v[1, pl.ds(c * LANES, LANES)] for c in range(CHUNKS)]
    seg2 = [segt_v[2, pl.ds(c * LANES, LANES)] for c in range(CHUNKS)]

    wb = []
    for q in range(NQ):
        gq[q].wait()

        def block(kk, _, q=q):
            lblv = lbl_v[q, pl.ds(kk * LANES, LANES)]
            for pi in range(LANES // BATCH):
                rr = kk * LANES + pi * BATCH
                p = (q * QROWS + rr) // BATCH
                pec = [pe_v[p, pl.ds(c * LANES, LANES)] for c in range(CHUNKS)]
                for b in range(BATCH):
                    i = pi * BATCH + b
                    r = q * QROWS + rr + b
                    lbl_b = lax.gather(
                        lblv,
                        jnp.full((LANES, 1), i, jnp.int32),
                        lax.GatherDimensionNumbers(
                            offset_dims=(), collapsed_slice_dims=(0,),
                            start_index_map=(0,)),
                        slice_sizes=(1,),
                        mode=lax.GatherScatterMode.PROMISE_IN_BOUNDS)
                    f1 = (lbl_b & 1).astype(jnp.float32)
                    f2 = (lbl_b >> 1).astype(jnp.float32)
                    for c in range(CHUNKS):
                        sl = pl.ds(c * LANES, LANES)
                        out_v[r, sl] = (tok_v[r, sl] * SCALE + pec[c]
                                        + f1 * seg1[c] + f2 * seg2[c])
            return _

        lax.fori_loop(0, QROWS // LANES, block, 0, unroll=False)

        wb.append(pltpu.async_copy(
            out_v.at[pl.ds(q * QROWS, QROWS)],
            out_hbm.at[pl.ds(base + q * QROWS, QROWS)], semw))

    for cp in wb:
        cp.wait()


def kernel(sequence, segment_label, token_table, segment_table, pe):
    seq2d = sequence.reshape(ROWS // QROWS, QROWS).astype(jnp.int32)
    lbl2d = segment_label.reshape(ROWS // QROWS, QROWS).astype(jnp.int32)
    pe2d = pe.reshape(pe.shape[0], EMBED)
    out = _sc_embed(seq2d, lbl2d, token_table, segment_table, pe2d)
    return out.reshape(SEQ_LEN, BATCH, EMBED)


# speedup vs baseline: 4.2694x; 1.1310x over previous
import math

import jax
import jax.numpy as jnp
from jax import lax
from jax.experimental import pallas as pl
from jax.experimental.pallas import tpu as pltpu
from jax.experimental.pallas import tpu_sc as plsc

VOCAB = 100000
EMBED = 128
SEQ_LEN = 2048
BATCH = 4
ROWS = SEQ_LEN * BATCH
NC, NS, LANES = 2, 16, 16
NW = NC * NS
RPW = ROWS // NW
PE_RPW = RPW // BATCH
CHUNKS = EMBED // LANES
NQ = 2
QROWS = RPW // NQ
SCALE = math.sqrt(EMBED)

_mesh = plsc.VectorSubcoreMesh(
    core_axis_name="c", subcore_axis_name="s", num_cores=NC, num_subcores=NS
)


@pl.kernel(
    out_type=jax.ShapeDtypeStruct((ROWS, EMBED), jnp.float32),
    mesh=_mesh,
    scratch_types=[
        pltpu.VMEM((NQ, QROWS), jnp.int32),
        pltpu.VMEM((NQ, QROWS), jnp.int32),
        pltpu.VMEM((RPW, EMBED), jnp.float32),
        pltpu.VMEM((RPW, EMBED), jnp.float32),
        pltpu.VMEM((4, EMBED), jnp.float32),
        pltpu.VMEM((PE_RPW, EMBED), jnp.float32),
        pltpu.SemaphoreType.DMA,
        pltpu.SemaphoreType.DMA,
        pltpu.SemaphoreType.DMA,
        pltpu.SemaphoreType.DMA,
    ],
)
def _sc_embed(seq_hbm, seg_hbm, tok_table, seg_table, pe_hbm, out_hbm,
              idx_v, lbl_v, tok_v, out_v, segt_v, pe_v,
              sems, semg0, semg1, semw):
    wid = lax.axis_index("s") * NC + lax.axis_index("c")
    base = wid * RPW
    semg = [semg0, semg1]

    cpi = pltpu.async_copy(seq_hbm.at[pl.ds(wid * NQ, NQ)], idx_v, sems)
    cpl = pltpu.async_copy(seg_hbm.at[pl.ds(wid * NQ, NQ)], lbl_v, sems)
    cpi.wait()

    gq = [pltpu.async_copy(tok_table.at[idx_v.at[q]],
                           tok_v.at[pl.ds(q * QROWS, QROWS)], semg[q])
          for q in range(NQ)]
    gs = pltpu.async_copy(seg_table.at[pl.ds(0, 3)], segt_v.at[pl.ds(0, 3)],
                          sems)
    gp = pltpu.async_copy(pe_hbm.at[pl.ds(wid * PE_RPW, PE_RPW)], pe_v, sems)
    cpl.wait(); gs.wait(); gp.wait()

    seg1 = [segt_v[1, pl.ds(c * LANES, LANES)] for c in range(CHUNKS)]
    seg2 = [segt_v[2, pl.ds(c * LANES, LANES)] for c in range(CHUNKS)]

    wb = []
    for q in range(NQ):
        gq[q].wait()

        def block(kk, _, q=q):
            lblv = lbl_v[q, pl.ds(kk * LANES, LANES)]
            for pi in range(LANES // BATCH):
                rr = kk * LANES + pi * BATCH
                p = (q * QROWS + rr) // BATCH
                pec = [pe_v[p, pl.ds(c * LANES, LANES)] for c in range(CHUNKS)]
                for b in range(BATCH):
                    i = pi * BATCH + b
                    r = q * QROWS + rr + b
                    lbl_b = lax.gather(
                        lblv,
                        jnp.full((LANES, 1), i, jnp.int32),
                        lax.GatherDimensionNumbers(
                            offset_dims=(), collapsed_slice_dims=(0,),
                            start_index_map=(0,)),
                        slice_sizes=(1,),
                        mode=lax.GatherScatterMode.PROMISE_IN_BOUNDS)
                    f1 = (lbl_b & 1).astype(jnp.float32)
                    f2 = (lbl_b >> 1).astype(jnp.float32)
                    for c in range(CHUNKS):
                        sl = pl.ds(c * LANES, LANES)
                        out_v[r, sl] = (tok_v[r, sl] * SCALE + pec[c]
                                        + f1 * seg1[c] + f2 * seg2[c])
            return _

        lax.fori_loop(0, QROWS // LANES, block, 0, unroll=False)

        wb.append(pltpu.async_copy(
            out_v.at[pl.ds(q * QROWS, QROWS)],
            out_hbm.at[pl.ds(base + q * QROWS, QROWS)], semw))

    for cp in wb:
        cp.wait()


def kernel(sequence, segment_label, token_table, segment_table, pe):
    seq2d = sequence.reshape(ROWS // QROWS, QROWS).astype(jnp.int32)
    lbl2d = segment_label.reshape(ROWS // QROWS, QROWS).astype(jnp.int32)
    pe2d = pe.reshape(pe.shape[0], EMBED)
    out = _sc_embed(seq2d, lbl2d, token_table, segment_table, pe2d)
    return out.reshape(SEQ_LEN, BATCH, EMBED)


# speedup vs baseline: 4.3132x; 1.0103x over previous
import math

import jax
import jax.numpy as jnp
from jax import lax
from jax.experimental import pallas as pl
from jax.experimental.pallas import tpu as pltpu
from jax.experimental.pallas import tpu_sc as plsc

VOCAB = 100000
EMBED = 128
SEQ_LEN = 2048
BATCH = 4
ROWS = SEQ_LEN * BATCH
NC, NS, LANES = 2, 16, 16
NW = NC * NS
RPW = ROWS // NW
PE_RPW = RPW // BATCH
CHUNKS = EMBED // LANES
NQ = 2
QROWS = RPW // NQ
SCALE = math.sqrt(EMBED)

_mesh = plsc.VectorSubcoreMesh(
    core_axis_name="c", subcore_axis_name="s", num_cores=NC, num_subcores=NS
)


@pl.kernel(
    out_type=jax.ShapeDtypeStruct((ROWS, EMBED), jnp.float32),
    mesh=_mesh,
    scratch_types=[
        pltpu.VMEM((NQ, QROWS), jnp.int32),
        pltpu.VMEM((NQ, QROWS), jnp.int32),
        pltpu.VMEM((RPW, EMBED), jnp.float32),
        pltpu.VMEM((RPW, EMBED), jnp.float32),
        pltpu.VMEM((4, EMBED), jnp.float32),
        pltpu.VMEM((PE_RPW, EMBED), jnp.float32),
        pltpu.SemaphoreType.DMA,
        pltpu.SemaphoreType.DMA,
        pltpu.SemaphoreType.DMA,
        pltpu.SemaphoreType.DMA,
    ],
)
def _sc_embed(seq_hbm, seg_hbm, tok_table, seg_table, pe_hbm, out_hbm,
              idx_v, lbl_v, tok_v, out_v, segt_v, pe_v,
              sems, semg0, semg1, semw):
    wid = lax.axis_index("s") * NC + lax.axis_index("c")
    base = wid * RPW
    semg = [semg0, semg1]

    cpi = pltpu.async_copy(seq_hbm.at[pl.ds(wid * NQ, NQ)], idx_v, sems)
    cpl = pltpu.async_copy(seg_hbm.at[pl.ds(wid * NQ, NQ)], lbl_v, sems)
    gs = pltpu.async_copy(seg_table.at[pl.ds(0, 3)], segt_v.at[pl.ds(0, 3)],
                          sems)
    gp = pltpu.async_copy(pe_hbm.at[pl.ds(wid * PE_RPW, PE_RPW)], pe_v, sems)
    cpi.wait()

    gq = [pltpu.async_copy(tok_table.at[idx_v.at[q]],
                           tok_v.at[pl.ds(q * QROWS, QROWS)], semg[q])
          for q in range(NQ)]
    cpl.wait(); gs.wait(); gp.wait()

    seg1 = [segt_v[1, pl.ds(c * LANES, LANES)] for c in range(CHUNKS)]
    seg2 = [segt_v[2, pl.ds(c * LANES, LANES)] for c in range(CHUNKS)]

    wb = []
    for q in range(NQ):
        gq[q].wait()

        def block(kk, _, q=q):
            lblv = lbl_v[q, pl.ds(kk * LANES, LANES)]
            for pi in range(LANES // BATCH):
                rr = kk * LANES + pi * BATCH
                p = (q * QROWS + rr) // BATCH
                pec = [pe_v[p, pl.ds(c * LANES, LANES)] for c in range(CHUNKS)]
                for b in range(BATCH):
                    i = pi * BATCH + b
                    r = q * QROWS + rr + b
                    lbl_b = lax.gather(
                        lblv,
                        jnp.full((LANES, 1), i, jnp.int32),
                        lax.GatherDimensionNumbers(
                            offset_dims=(), collapsed_slice_dims=(0,),
                            start_index_map=(0,)),
                        slice_sizes=(1,),
                        mode=lax.GatherScatterMode.PROMISE_IN_BOUNDS)
                    f1 = (lbl_b & 1).astype(jnp.float32)
                    f2 = (lbl_b >> 1).astype(jnp.float32)
                    for c in range(CHUNKS):
                        sl = pl.ds(c * LANES, LANES)
                        out_v[r, sl] = (tok_v[r, sl] * SCALE + pec[c]
                                        + f1 * seg1[c] + f2 * seg2[c])
            return _

        lax.fori_loop(0, QROWS // LANES, block, 0, unroll=False)

        wb.append(pltpu.async_copy(
            out_v.at[pl.ds(q * QROWS, QROWS)],
            out_hbm.at[pl.ds(base + q * QROWS, QROWS)], semw))

    for cp in wb:
        cp.wait()


def kernel(sequence, segment_label, token_table, segment_table, pe):
    seq2d = sequence.reshape(ROWS // QROWS, QROWS).astype(jnp.int32)
    lbl2d = segment_label.reshape(ROWS // QROWS, QROWS).astype(jnp.int32)
    pe2d = pe.reshape(pe.shape[0], EMBED)
    out = _sc_embed(seq2d, lbl2d, token_table, segment_table, pe2d)
    return out.reshape(SEQ_LEN, BATCH, EMBED)


# speedup vs baseline: 4.3188x; 1.0013x over previous
import math

import jax
import jax.numpy as jnp
from jax import lax
from jax.experimental import pallas as pl
from jax.experimental.pallas import tpu as pltpu
from jax.experimental.pallas import tpu_sc as plsc

VOCAB = 100000
EMBED = 128
SEQ_LEN = 2048
BATCH = 4
ROWS = SEQ_LEN * BATCH
NC, NS, LANES = 2, 16, 16
NW = NC * NS
RPW = ROWS // NW
LPW = RPW // BATCH
CHUNKS = EMBED // LANES
NQ = 2
QROWS = RPW // NQ
SCALE = math.sqrt(EMBED)

_mesh = plsc.VectorSubcoreMesh(
    core_axis_name="c", subcore_axis_name="s", num_cores=NC, num_subcores=NS
)


@pl.kernel(
    out_type=jax.ShapeDtypeStruct((ROWS, EMBED), jnp.float32),
    mesh=_mesh,
    scratch_types=[
        pltpu.VMEM((RPW,), jnp.int32),
        pltpu.VMEM((RPW,), jnp.int32),
        pltpu.VMEM((RPW, EMBED), jnp.float32),
        pltpu.VMEM((RPW, EMBED), jnp.float32),
        pltpu.VMEM((4, EMBED), jnp.float32),
        pltpu.VMEM((LPW, EMBED), jnp.float32),
        pltpu.SemaphoreType.DMA,
        pltpu.SemaphoreType.DMA,
        pltpu.SemaphoreType.DMA,
        pltpu.SemaphoreType.DMA,
    ],
)
def _sc_embed(seq_hbm, seg_hbm, tok_table, seg_table, pe_hbm, out_hbm,
              idx_v, lbl_v, tok_v, out_v, segt_v, pe_v,
              sems, semg0, semg1, semw):
    wid = lax.axis_index("s") * NC + lax.axis_index("c")
    base = wid * RPW
    semg = [semg0, semg1]

    cpi = pltpu.async_copy(seq_hbm.at[pl.ds(base, RPW)], idx_v, sems)
    cpl = pltpu.async_copy(seg_hbm.at[pl.ds(base, RPW)], lbl_v, sems)
    gs = pltpu.async_copy(seg_table.at[pl.ds(0, 3)], segt_v.at[pl.ds(0, 3)],
                          sems)
    gp = pltpu.async_copy(pe_hbm.at[pl.ds(wid * LPW, LPW)], pe_v, sems)
    cpi.wait(); cpl.wait()

    gq = [pltpu.async_copy(tok_table.at[idx_v.at[pl.ds(q * QROWS, QROWS)]],
                           tok_v.at[pl.ds(q * QROWS, QROWS)], semg[q])
          for q in range(NQ)]
    gs.wait(); gp.wait()

    seg1 = [segt_v[1, pl.ds(c * LANES, LANES)] for c in range(CHUNKS)]
    seg2 = [segt_v[2, pl.ds(c * LANES, LANES)] for c in range(CHUNKS)]

    wb = []
    for q in range(NQ):
        gq[q].wait()

        def block(kk, _, q=q):
            lblv = lbl_v[pl.ds(q * QROWS + kk * LANES, LANES)]
            for pi in range(LANES // BATCH):
                rr = kk * LANES + pi * BATCH
                p = (q * QROWS + rr) // BATCH
                pec = [pe_v[p, pl.ds(c * LANES, LANES)] for c in range(CHUNKS)]
                for b in range(BATCH):
                    i = pi * BATCH + b
                    r = q * QROWS + rr + b
                    lbl_b = lax.gather(
                        lblv,
                        jnp.full((LANES, 1), i, jnp.int32),
                        lax.GatherDimensionNumbers(
                            offset_dims=(), collapsed_slice_dims=(0,),
                            start_index_map=(0,)),
                        slice_sizes=(1,),
                        mode=lax.GatherScatterMode.PROMISE_IN_BOUNDS)
                    f1 = (lbl_b & 1).astype(jnp.float32)
                    f2 = (lbl_b >> 1).astype(jnp.float32)
                    for c in range(CHUNKS):
                        sl = pl.ds(c * LANES, LANES)
                        out_v[r, sl] = (tok_v[r, sl] * SCALE + pec[c]
                                        + f1 * seg1[c] + f2 * seg2[c])
            return _

        lax.fori_loop(0, QROWS // LANES, block, 0, unroll=False)

        wb.append(pltpu.async_copy(
            out_v.at[pl.ds(q * QROWS, QROWS)],
            out_hbm.at[pl.ds(base + q * QROWS, QROWS)], semw))

    for cp in wb:
        cp.wait()


def kernel(sequence, segment_label, token_table, segment_table, pe):
    seq1d = sequence.reshape(ROWS).astype(jnp.int32)
    lbl1d = segment_label.reshape(ROWS).astype(jnp.int32)
    pe2d = pe.reshape(pe.shape[0], EMBED)
    out = _sc_embed(seq1d, lbl1d, token_table, segment_table, pe2d)
    return out.reshape(SEQ_LEN, BATCH, EMBED)


# speedup vs baseline: 4.3557x; 1.0086x over previous
import math

import jax
import jax.numpy as jnp
from jax import lax
from jax.experimental import pallas as pl
from jax.experimental.pallas import tpu as pltpu
from jax.experimental.pallas import tpu_sc as plsc

VOCAB = 100000
EMBED = 128
SEQ_LEN = 2048
BATCH = 4
ROWS = SEQ_LEN * BATCH
NC, NS, LANES = 2, 16, 16
NW = NC * NS
RPW = ROWS // NW
LPW = RPW // BATCH
CHUNKS = EMBED // LANES
NQ = 2
QROWS = RPW // NQ
SCALE = math.sqrt(EMBED)

_mesh = plsc.VectorSubcoreMesh(
    core_axis_name="c", subcore_axis_name="s", num_cores=NC, num_subcores=NS
)


@pl.kernel(
    out_type=jax.ShapeDtypeStruct((ROWS, EMBED), jnp.float32),
    mesh=_mesh,
    scratch_types=[
        pltpu.VMEM((RPW,), jnp.int32),
        pltpu.VMEM((RPW,), jnp.int32),
        pltpu.VMEM((RPW, EMBED), jnp.float32),
        pltpu.VMEM((RPW, EMBED), jnp.float32),
        pltpu.VMEM((4, EMBED), jnp.float32),
        pltpu.VMEM((LPW, EMBED), jnp.float32),
        pltpu.SemaphoreType.DMA,
        pltpu.SemaphoreType.DMA,
        pltpu.SemaphoreType.DMA,
        pltpu.SemaphoreType.DMA,
    ],
)
def _sc_embed(ids_hbm, tok_table, seg_table, pe_hbm, out_hbm,
              idx_v, lbl_v, tok_v, out_v, segt_v, pe_v,
              sems, semg0, semg1, semw):
    wid = lax.axis_index("s") * NC + lax.axis_index("c")
    base = wid * RPW
    semg = [semg0, semg1]

    cpi = pltpu.async_copy(ids_hbm.at[0, pl.ds(base, RPW)], idx_v, sems)
    cpl = pltpu.async_copy(ids_hbm.at[1, pl.ds(base, RPW)], lbl_v, sems)
    gs = pltpu.async_copy(seg_table.at[pl.ds(0, 3)], segt_v.at[pl.ds(0, 3)],
                          sems)
    gp = pltpu.async_copy(pe_hbm.at[pl.ds(wid * LPW, LPW)], pe_v, sems)
    cpi.wait(); cpl.wait()

    gq = [pltpu.async_copy(tok_table.at[idx_v.at[pl.ds(q * QROWS, QROWS)]],
                           tok_v.at[pl.ds(q * QROWS, QROWS)], semg[q])
          for q in range(NQ)]
    gs.wait(); gp.wait()

    seg1 = [segt_v[1, pl.ds(c * LANES, LANES)] for c in range(CHUNKS)]
    seg2 = [segt_v[2, pl.ds(c * LANES, LANES)] for c in range(CHUNKS)]

    wb = []
    for q in range(NQ):
        gq[q].wait()

        def block(kk, _, q=q):
            lblv = lbl_v[pl.ds(q * QROWS + kk * LANES, LANES)]
            for pi in range(LANES // BATCH):
                rr = kk * LANES + pi * BATCH
                p = (q * QROWS + rr) // BATCH
                pec = [pe_v[p, pl.ds(c * LANES, LANES)] for c in range(CHUNKS)]
                for b in range(BATCH):
                    i = pi * BATCH + b
                    r = q * QROWS + rr + b
                    lbl_b = lax.gather(
                        lblv,
                        jnp.full((LANES, 1), i, jnp.int32),
                        lax.GatherDimensionNumbers(
                            offset_dims=(), collapsed_slice_dims=(0,),
                            start_index_map=(0,)),
                        slice_sizes=(1,),
                        mode=lax.GatherScatterMode.PROMISE_IN_BOUNDS)
                    f1 = (lbl_b & 1).astype(jnp.float32)
                    f2 = (lbl_b >> 1).astype(jnp.float32)
                    for c in range(CHUNKS):
                        sl = pl.ds(c * LANES, LANES)
                        out_v[r, sl] = (tok_v[r, sl] * SCALE + pec[c]
                                        + f1 * seg1[c] + f2 * seg2[c])
            return _

        lax.fori_loop(0, QROWS // LANES, block, 0, unroll=False)

        wb.append(pltpu.async_copy(
            out_v.at[pl.ds(q * QROWS, QROWS)],
            out_hbm.at[pl.ds(base + q * QROWS, QROWS)], semw))

    for cp in wb:
        cp.wait()


def kernel(sequence, segment_label, token_table, segment_table, pe):
    ids = jnp.stack([sequence, segment_label]).reshape(2, ROWS)
    ids = ids.astype(jnp.int32)
    pe2d = pe.reshape(pe.shape[0], EMBED)
    out = _sc_embed(ids, token_table, segment_table, pe2d)
    return out.reshape(SEQ_LEN, BATCH, EMBED)
